# baseline (speedup 1.0000x reference)
import functools
import math

import jax
import jax.numpy as jnp
import numpy as np
from jax import lax
from jax.experimental import pallas as pl
from jax.experimental.pallas import tpu as pltpu
from jax.experimental.pallas import tpu_sc as plsc

_N = 65536
_NW = 32
_PW = 2048


def _brev8(j):
    j = np.asarray(j)
    r = np.zeros_like(j)
    t = j.copy()
    for _ in range(8):
        r = (r << 1) | (t & 1)
        t >>= 1
    return r


_P = np.arange(_PW)
_C8 = _P % 8
_J = _P // 8
_W = np.arange(_NW)[:, None]
_n1 = 8 * _W + _C8[None, :]

_IDX1R = (2 * (_n1 + 256 * _brev8(_J)[None, :])).astype(np.int32)
_IDX1I = _IDX1R + 1

_n1b = _brev8(_J)[None, :] + 0 * _W
_k2 = 8 * _W + _C8[None, :]
_IDX2 = ((_n1b >> 3) * 2048 + _k2 * 8 + (_n1b & 7)).astype(np.int32)

_OIDXR = (2 * (8 * _W + _C8[None, :] + 256 * _J[None, :])).astype(np.int32)
_OIDXI = _OIDXR + 1

_TWR = np.zeros((7, 1024), np.float32)
_TWI = np.zeros((7, 1024), np.float32)
for _si in range(7):
    _h = 2 << _si
    _q = np.arange(8 * _h)
    _ang = -2.0 * np.pi * (_q // 8) / (2 * _h)
    _TWR[_si, :8 * _h] = np.cos(_ang)
    _TWI[_si, :8 * _h] = np.sin(_ang)

_ang2 = -2.0 * np.pi * (_n1 * _J[None, :]) / _N
_T2R = np.cos(_ang2).astype(np.float32)
_T2I = np.sin(_ang2).astype(np.float32)

_jIDX1R = _IDX1R.reshape(_NW, 16, 128)
_jIDX1I = _IDX1I.reshape(_NW, 16, 128)
_jIDX2 = _IDX2.reshape(_NW, 16, 128)
_jOIDXR = _OIDXR.reshape(_NW, 16, 128)
_jOIDXI = _OIDXI.reshape(_NW, 16, 128)
_jTWR = _TWR
_jTWI = _TWI
_jT2R = _T2R
_jT2I = _T2I

_mesh = plsc.VectorSubcoreMesh(core_axis_name="c", subcore_axis_name="s")


def _wid():
    return lax.axis_index("s") * 2 + lax.axis_index("c")


def _fft_stages(br, bi, twr, twi):
    lanes = lax.iota(jnp.int32, 16)
    perm = lanes ^ 8
    topm = lanes < 8

    def s0(t, acc):
        a = t * 16
        vr = br[pl.ds(a, 16)]
        vi = bi[pl.ds(a, 16)]
        ur = vr.at[perm].get(mode="promise_in_bounds", unique_indices=True)
        ui = vi.at[perm].get(mode="promise_in_bounds", unique_indices=True)
        br[pl.ds(a, 16)] = jnp.where(topm, vr + ur, ur - vr)
        bi[pl.ds(a, 16)] = jnp.where(topm, vi + ui, ui - vi)
        return acc

    lax.fori_loop(0, 128, s0, 0)

    for si in range(7):
        h = 2 << si

        def body(t, acc, si=si, h=h):
            g = t >> si
            v = t & (h // 2 - 1)
            a = g * (16 * h) + 16 * v
            b = a + 8 * h
            wr = twr[si, pl.ds(16 * v, 16)]
            wi = twi[si, pl.ds(16 * v, 16)]
            tr = br[pl.ds(a, 16)]
            ti = bi[pl.ds(a, 16)]
            zr = br[pl.ds(b, 16)]
            zi = bi[pl.ds(b, 16)]
            pr = wr * zr - wi * zi
            pi = wr * zi + wi * zr
            br[pl.ds(a, 16)] = tr + pr
            bi[pl.ds(a, 16)] = ti + pi
            br[pl.ds(b, 16)] = tr - pr
            bi[pl.ds(b, 16)] = ti - pi
            return acc

        lax.fori_loop(0, 64, body, 0)


@functools.partial(
    pl.kernel,
    mesh=_mesh,
    out_type=[jax.ShapeDtypeStruct((_N,), jnp.float32)] * 2,
    scratch_types=[
        pltpu.VMEM((16, 128), jnp.int32),
        pltpu.VMEM((16, 128), jnp.int32),
        pltpu.VMEM((_PW,), jnp.float32),
        pltpu.VMEM((_PW,), jnp.float32),
        pltpu.VMEM((7, 1024), jnp.float32),
        pltpu.VMEM((7, 1024), jnp.float32),
        pltpu.VMEM((_PW,), jnp.float32),
        pltpu.VMEM((_PW,), jnp.float32),
        pltpu.SemaphoreType.DMA,
        pltpu.SemaphoreType.DMA,
    ],
)
def _pass1(xflat, idx1r, idx1i, twr_h, twi_h, t2r_h, t2i_h,
           cr_h, ci_h,
           idxr, idxi, br, bi, twr, twi, t2r, t2i, sem_g, sem_t):
    w = _wid()
    cs = [
        pltpu.async_copy(twr_h, twr, sem_t),
        pltpu.async_copy(twi_h, twi, sem_t),
        pltpu.async_copy(t2r_h.at[w], t2r, sem_t),
        pltpu.async_copy(t2i_h.at[w], t2i, sem_t),
    ]
    pltpu.sync_copy(idx1r.at[w], idxr)
    pltpu.sync_copy(idx1i.at[w], idxi)
    gs = []
    for j in range(16):
        gs.append(pltpu.async_copy(
            xflat.at[idxr.at[j]], br.at[pl.ds(j * 128, 128)], sem_g))
        gs.append(pltpu.async_copy(
            xflat.at[idxi.at[j]], bi.at[pl.ds(j * 128, 128)], sem_g))
    for c in cs:
        c.wait()
    for c in gs:
        c.wait()
    _fft_stages(br, bi, twr, twi)

    def twid(t, acc):
        a = t * 16
        vr = br[pl.ds(a, 16)]
        vi = bi[pl.ds(a, 16)]
        fr = t2r[pl.ds(a, 16)]
        fi = t2i[pl.ds(a, 16)]
        br[pl.ds(a, 16)] = vr * fr - vi * fi
        bi[pl.ds(a, 16)] = vr * fi + vi * fr
        return acc

    lax.fori_loop(0, 128, twid, 0)
    pltpu.sync_copy(br, cr_h.at[pl.ds(w * _PW, _PW)])
    pltpu.sync_copy(bi, ci_h.at[pl.ds(w * _PW, _PW)])


@functools.partial(
    pl.kernel,
    mesh=_mesh,
    out_type=jax.ShapeDtypeStruct((2 * _N,), jnp.float32),
    scratch_types=[
        pltpu.VMEM((16, 128), jnp.int32),
        pltpu.VMEM((16, 128), jnp.int32),
        pltpu.VMEM((16, 128), jnp.int32),
        pltpu.VMEM((_PW,), jnp.float32),
        pltpu.VMEM((_PW,), jnp.float32),
        pltpu.VMEM((7, 1024), jnp.float32),
        pltpu.VMEM((7, 1024), jnp.float32),
        pltpu.SemaphoreType.DMA,
        pltpu.SemaphoreType.DMA,
    ],
)
def _pass2(cr_h, ci_h, idx2, oidxr_h, oidxi_h, twr_h, twi_h,
           outflat,
           idxg, oidxr, oidxi, br, bi, twr, twi, sem_g, sem_t):
    w = _wid()
    cs = [
        pltpu.async_copy(twr_h, twr, sem_t),
        pltpu.async_copy(twi_h, twi, sem_t),
        pltpu.async_copy(oidxr_h.at[w], oidxr, sem_t),
        pltpu.async_copy(oidxi_h.at[w], oidxi, sem_t),
    ]
    pltpu.sync_copy(idx2.at[w], idxg)
    gs = []
    for j in range(16):
        gs.append(pltpu.async_copy(
            cr_h.at[idxg.at[j]], br.at[pl.ds(j * 128, 128)], sem_g))
        gs.append(pltpu.async_copy(
            ci_h.at[idxg.at[j]], bi.at[pl.ds(j * 128, 128)], sem_g))
    for c in cs:
        c.wait()
    for c in gs:
        c.wait()
    _fft_stages(br, bi, twr, twi)
    ss = []
    for j in range(16):
        ss.append(pltpu.async_copy(
            br.at[pl.ds(j * 128, 128)], outflat.at[oidxr.at[j]], sem_g))
        ss.append(pltpu.async_copy(
            bi.at[pl.ds(j * 128, 128)], outflat.at[oidxi.at[j]], sem_g))
    for c in ss:
        c.wait()


def kernel(x):
    xflat = x.reshape(2 * _N)
    cr, ci = _pass1(xflat, _jIDX1R, _jIDX1I, _jTWR, _jTWI, _jT2R, _jT2I)
    outflat = _pass2(cr, ci, _jIDX2, _jOIDXR, _jOIDXI, _jTWR, _jTWI)
    return outflat.reshape(_N, 2)


# speedup vs baseline: 3.0142x; 3.0142x over previous
import functools
import math

import jax
import jax.numpy as jnp
import numpy as np
from jax import lax
from jax.experimental import pallas as pl
from jax.experimental.pallas import tpu as pltpu
from jax.experimental.pallas import tpu_sc as plsc

_N = 65536
_NW = 32
_PW = 2048


def _brev8(j):
    j = np.asarray(j)
    r = np.zeros_like(j)
    t = j.copy()
    for _ in range(8):
        r = (r << 1) | (t & 1)
        t >>= 1
    return r


_P = np.arange(_PW)
_C8 = _P % 8
_J = _P // 8
_W = np.arange(_NW)[:, None]
_n1 = 8 * _W + _C8[None, :]

_IDX1R = (2 * (_n1 + 256 * _brev8(_J)[None, :])).astype(np.int32)
_IDX1I = _IDX1R + 1

_n1b = _brev8(_J)[None, :] + 0 * _W
_k2 = 8 * _W + _C8[None, :]
_IDX2 = ((_n1b >> 3) * 2048 + _k2 * 8 + (_n1b & 7)).astype(np.int32)


_TWR = np.zeros((7, 1024), np.float32)
_TWI = np.zeros((7, 1024), np.float32)
for _si in range(7):
    _h = 2 << _si
    _q = np.arange(8 * _h)
    _ang = -2.0 * np.pi * (_q // 8) / (2 * _h)
    _TWR[_si, :8 * _h] = np.cos(_ang)
    _TWI[_si, :8 * _h] = np.sin(_ang)

_ang2 = -2.0 * np.pi * (_n1 * _J[None, :]) / _N
_T2R = np.cos(_ang2).astype(np.float32)
_T2I = np.sin(_ang2).astype(np.float32)

_jIDX1R = _IDX1R.reshape(_NW, 16, 128)
_jIDX1I = _IDX1I.reshape(_NW, 16, 128)
_jIDX2 = _IDX2.reshape(_NW, 16, 128)
_jTWR = _TWR
_jTWI = _TWI
_jT2R = _T2R
_jT2I = _T2I

_mesh = plsc.VectorSubcoreMesh(core_axis_name="c", subcore_axis_name="s")


def _wid():
    return lax.axis_index("s") * 2 + lax.axis_index("c")


def _fft_stages(br, bi, twr, twi):
    lanes = lax.iota(jnp.int32, 16)
    perm = lanes ^ 8
    topm = lanes < 8

    def s0(t, acc):
        a = t * 16
        vr = br[pl.ds(a, 16)]
        vi = bi[pl.ds(a, 16)]
        ur = vr.at[perm].get(mode="promise_in_bounds", unique_indices=True)
        ui = vi.at[perm].get(mode="promise_in_bounds", unique_indices=True)
        br[pl.ds(a, 16)] = jnp.where(topm, vr + ur, ur - vr)
        bi[pl.ds(a, 16)] = jnp.where(topm, vi + ui, ui - vi)
        return acc

    lax.fori_loop(0, 128, s0, 0)

    for si in range(7):
        h = 2 << si

        def body(t, acc, si=si, h=h):
            g = t >> si
            v = t & (h // 2 - 1)
            a = g * (16 * h) + 16 * v
            b = a + 8 * h
            wr = twr[si, pl.ds(16 * v, 16)]
            wi = twi[si, pl.ds(16 * v, 16)]
            tr = br[pl.ds(a, 16)]
            ti = bi[pl.ds(a, 16)]
            zr = br[pl.ds(b, 16)]
            zi = bi[pl.ds(b, 16)]
            pr = wr * zr - wi * zi
            pi = wr * zi + wi * zr
            br[pl.ds(a, 16)] = tr + pr
            bi[pl.ds(a, 16)] = ti + pi
            br[pl.ds(b, 16)] = tr - pr
            bi[pl.ds(b, 16)] = ti - pi
            return acc

        lax.fori_loop(0, 64, body, 0)


@functools.partial(
    pl.kernel,
    mesh=_mesh,
    out_type=[jax.ShapeDtypeStruct((_N,), jnp.float32)] * 2,
    scratch_types=[
        pltpu.VMEM((16, 128), jnp.int32),
        pltpu.VMEM((16, 128), jnp.int32),
        pltpu.VMEM((_PW,), jnp.float32),
        pltpu.VMEM((_PW,), jnp.float32),
        pltpu.VMEM((7, 1024), jnp.float32),
        pltpu.VMEM((7, 1024), jnp.float32),
        pltpu.VMEM((_PW,), jnp.float32),
        pltpu.VMEM((_PW,), jnp.float32),
        pltpu.SemaphoreType.DMA,
        pltpu.SemaphoreType.DMA,
    ],
)
def _pass1(xflat, idx1r, idx1i, twr_h, twi_h, t2r_h, t2i_h,
           cr_h, ci_h,
           idxr, idxi, br, bi, twr, twi, t2r, t2i, sem_g, sem_t):
    w = _wid()
    cs = [
        pltpu.async_copy(twr_h, twr, sem_t),
        pltpu.async_copy(twi_h, twi, sem_t),
        pltpu.async_copy(t2r_h.at[w], t2r, sem_t),
        pltpu.async_copy(t2i_h.at[w], t2i, sem_t),
    ]
    pltpu.sync_copy(idx1r.at[w], idxr)
    pltpu.sync_copy(idx1i.at[w], idxi)
    gs = []
    for j in range(16):
        gs.append(pltpu.async_copy(
            xflat.at[idxr.at[j]], br.at[pl.ds(j * 128, 128)], sem_g))
        gs.append(pltpu.async_copy(
            xflat.at[idxi.at[j]], bi.at[pl.ds(j * 128, 128)], sem_g))
    for c in cs:
        c.wait()
    for c in gs:
        c.wait()
    _fft_stages(br, bi, twr, twi)

    def twid(t, acc):
        a = t * 16
        vr = br[pl.ds(a, 16)]
        vi = bi[pl.ds(a, 16)]
        fr = t2r[pl.ds(a, 16)]
        fi = t2i[pl.ds(a, 16)]
        br[pl.ds(a, 16)] = vr * fr - vi * fi
        bi[pl.ds(a, 16)] = vr * fi + vi * fr
        return acc

    lax.fori_loop(0, 128, twid, 0)
    pltpu.sync_copy(br, cr_h.at[pl.ds(w * _PW, _PW)])
    pltpu.sync_copy(bi, ci_h.at[pl.ds(w * _PW, _PW)])


@functools.partial(
    pl.kernel,
    mesh=_mesh,
    out_type=jax.ShapeDtypeStruct((256, 32, 16), jnp.float32),
    scratch_types=[
        pltpu.VMEM((16, 128), jnp.int32),
        pltpu.VMEM((_PW,), jnp.float32),
        pltpu.VMEM((_PW,), jnp.float32),
        pltpu.VMEM((256, 16), jnp.float32),
        pltpu.VMEM((7, 1024), jnp.float32),
        pltpu.VMEM((7, 1024), jnp.float32),
        pltpu.SemaphoreType.DMA,
        pltpu.SemaphoreType.DMA,
    ],
)
def _pass2(cr_h, ci_h, idx2, twr_h, twi_h,
           out2d,
           idxg, br, bi, pk, twr, twi, sem_g, sem_t):
    w = _wid()
    cs = [
        pltpu.async_copy(twr_h, twr, sem_t),
        pltpu.async_copy(twi_h, twi, sem_t),
    ]
    pltpu.sync_copy(idx2.at[w], idxg)
    gs = []
    for j in range(16):
        gs.append(pltpu.async_copy(
            cr_h.at[idxg.at[j]], br.at[pl.ds(j * 128, 128)], sem_g))
        gs.append(pltpu.async_copy(
            ci_h.at[idxg.at[j]], bi.at[pl.ds(j * 128, 128)], sem_g))
    for c in cs:
        c.wait()
    for c in gs:
        c.wait()
    _fft_stages(br, bi, twr, twi)

    lanes = lax.iota(jnp.int32, 16)
    half = lanes >> 1
    par = lanes & 1

    def packbody(tp, acc):
        vr = br[pl.ds(16 * tp, 16)]
        vi = bi[pl.ds(16 * tp, 16)]
        g0r = vr.at[half].get(mode="promise_in_bounds")
        g0i = vi.at[half].get(mode="promise_in_bounds")
        g1r = vr.at[half + 8].get(mode="promise_in_bounds")
        g1i = vi.at[half + 8].get(mode="promise_in_bounds")
        pk[2 * tp, :] = jnp.where(par == 0, g0r, g0i)
        pk[2 * tp + 1, :] = jnp.where(par == 0, g1r, g1i)
        return acc

    lax.fori_loop(0, 128, packbody, 0)
    pltpu.sync_copy(pk, out2d.at[:, w])


def kernel(x):
    xflat = x.reshape(2 * _N)
    cr, ci = _pass1(xflat, _jIDX1R, _jIDX1I, _jTWR, _jTWI, _jT2R, _jT2I)
    out2d = _pass2(cr, ci, _jIDX2, _jTWR, _jTWI)
    return out2d.reshape(_N, 2)
